# baseline (speedup 1.0000x reference)
import jax
import jax.numpy as jnp
from jax import lax
from jax.experimental import pallas as pl
from jax.experimental.pallas import tpu as pltpu
from jax.experimental.pallas import tpu_sc as plsc

_VOCAB = 1000000
_DIM = 32
_BATCH = 16384
_NEG = 5
_ALPHA = 0.025

_NC = 2
_NS = 16
_NW = _NC * _NS
_CPW = _BATCH // _NW
_CPS = _BATCH // _NS
_DUMP = _BATCH
_NDUMP = 1024

_f32 = jnp.float32
_i32 = jnp.int32


def _gather_body(ev, ec, u2, p2, n3, U, P, N, idxb, rowb):
  wid = lax.axis_index("s") * _NC + lax.axis_index("c")
  r0 = wid * (_CPW // 128)
  b0 = wid * _CPW

  def fetch(tbl, idx_src, out_dst):
    pltpu.sync_copy(idx_src, idxb)
    for j in range(_CPW // 128):
      pltpu.sync_copy(tbl.at[idxb.at[j]], rowb.at[pl.ds(j * 128, 128)])
    pltpu.sync_copy(rowb, out_dst)

  fetch(ev, u2.at[pl.ds(r0, _CPW // 128)], U.at[pl.ds(b0, _CPW)])
  fetch(ec, p2.at[pl.ds(r0, _CPW // 128)], P.at[pl.ds(b0, _CPW)])
  for k in range(_NEG):
    fetch(ec, n3.at[k, pl.ds(r0, _CPW // 128)], N.at[k, pl.ds(b0, _CPW)])


_gather_call = pl.kernel(
    _gather_body,
    out_type=(
        jax.ShapeDtypeStruct((_BATCH, _DIM), _f32),
        jax.ShapeDtypeStruct((_BATCH, _DIM), _f32),
        jax.ShapeDtypeStruct((_NEG, _BATCH, _DIM), _f32),
    ),
    mesh=plsc.VectorSubcoreMesh(core_axis_name="c", subcore_axis_name="s",
                                num_cores=_NC, num_subcores=_NS),
    scratch_types=[
        pltpu.VMEM((_CPW // 128, 128), _i32),
        pltpu.VMEM((_CPW, _DIM), _f32),
    ],
    compiler_params=pltpu.CompilerParams(use_tc_tiling_on_sc=False),
)


def _err_body(u_ref, p_ref, n_ref, e_ref):
  uu = u_ref[...]
  pp = p_ref[...]
  sp = jnp.sum(uu * pp, axis=1)
  e = (_ALPHA * (1.0 - jax.nn.sigmoid(sp)))[:, None] * pp
  for k in range(_NEG):
    nk = n_ref[k]
    sk = jnp.sum(uu * nk, axis=1)
    e = e - (_ALPHA * jax.nn.sigmoid(sk))[:, None] * nk
  e_ref[...] = e


_BB = 2048

_err_call = pl.pallas_call(
    _err_body,
    grid=(_BATCH // _BB,),
    in_specs=[
        pl.BlockSpec((_BB, _DIM), lambda i: (i, 0)),
        pl.BlockSpec((_BB, _DIM), lambda i: (i, 0)),
        pl.BlockSpec((_NEG, _BB, _DIM), lambda i: (0, i, 0)),
    ],
    out_specs=pl.BlockSpec((_BB, _DIM), lambda i: (i, 0)),
    out_shape=jax.ShapeDtypeStruct((_BATCH, _DIM), _f32),
)


def _scatter_body(out_hbm, err_hbm, u2, bv2, ev,
                  idxb, bvb, wb, wsc, errb, rowb, delta):
  wid = lax.axis_index("s")
  r0 = wid * (_CPS // 128)
  b0 = wid * _CPS
  nj = _CPS // 128
  io = lax.broadcasted_iota(_i32, (16,), 0)
  zero = jnp.zeros((16,), _i32)

  pltpu.sync_copy(u2.at[pl.ds(r0, nj)], idxb)
  pltpu.sync_copy(bv2.at[pl.ds(r0, nj)], bvb)
  pltpu.sync_copy(err_hbm.at[pl.ds(b0, _CPS)], errb)
  pltpu.sync_copy(errb, delta.at[pl.ds(b0, _CPS)])

  for j in range(nj):
    for cc in range(8):
      rid = io + (j * 128 + cc * 16)
      bvv = bvb[j, pl.ds(cc * 16, 16)]
      plsc.store_scatter(rowb, [rid, zero], plsc.bitcast(bvv, _f32))
  for j in range(nj):
    pltpu.sync_copy(rowb.at[pl.ds(j * 128, 128)], out_hbm.at[idxb.at[j]])

  plsc.subcore_barrier()

  for j in range(nj):
    pltpu.sync_copy(out_hbm.at[idxb.at[j]], rowb.at[pl.ds(j * 128, 128)])
  for j in range(nj):
    for cc in range(8):
      sl = pl.ds(cc * 16, 16)
      rid = io + (j * 128 + cc * 16)
      wv = plsc.bitcast(plsc.load_gather(rowb, [rid, zero]), _i32)
      bvv = bvb[j, sl]
      wb[j, sl] = wv
      wsc[j, sl] = jnp.where(wv == bvv, _DUMP + (bvv & (_NDUMP - 1)), wv)
  for j in range(nj):
    pltpu.sync_copy(errb.at[pl.ds(j * 128, 128)], delta.at[wsc.at[j]],
                    add=True)
  for j in range(nj):
    pltpu.sync_copy(ev.at[idxb.at[j]], rowb.at[pl.ds(j * 128, 128)])
  for j in range(nj):
    pltpu.sync_copy(rowb.at[pl.ds(j * 128, 128)], delta.at[bvb.at[j]],
                    add=True)

  plsc.subcore_barrier()

  for j in range(nj):
    pltpu.sync_copy(delta.at[wb.at[j]], errb.at[pl.ds(j * 128, 128)])
  for j in range(nj):
    pltpu.sync_copy(errb.at[pl.ds(j * 128, 128)], out_hbm.at[idxb.at[j]])


_scatter_call = pl.kernel(
    _scatter_body,
    out_type=(),
    mesh=plsc.VectorSubcoreMesh(core_axis_name="c", subcore_axis_name="s",
                                num_cores=1, num_subcores=_NS),
    scratch_types=[
        pltpu.VMEM((_CPS // 128, 128), _i32),
        pltpu.VMEM((_CPS // 128, 128), _i32),
        pltpu.VMEM((_CPS // 128, 128), _i32),
        pltpu.VMEM((_CPS // 128, 128), _i32),
        pltpu.VMEM((_CPS, _DIM), _f32),
        pltpu.VMEM((_CPS, _DIM), _f32),
        pltpu.VMEM_SHARED((_BATCH + _NDUMP, _DIM), _f32),
    ],
    compiler_params=pltpu.CompilerParams(use_tc_tiling_on_sc=False,
                                         needs_layout_passes=False),
)


def kernel(emb_vertex, emb_context, u, pos_v, neg_v):
  u2 = u.reshape(128, 128)
  p2 = pos_v.reshape(128, 128)
  n3 = neg_v.T.reshape(_NEG, 128, 128)
  bv2 = jnp.arange(_BATCH, dtype=_i32).reshape(128, 128)

  U, P, N = _gather_call(emb_vertex, emb_context, u2, p2, n3)
  err = _err_call(U, P, N)

  out_ref = jax.new_ref(emb_vertex)
  _scatter_call(out_ref, err, u2, bv2, emb_vertex)
  return out_ref[...]


# speedup vs baseline: 1.6301x; 1.6301x over previous
import jax
import jax.numpy as jnp
from jax import lax
from jax.experimental import pallas as pl
from jax.experimental.pallas import tpu as pltpu
from jax.experimental.pallas import tpu_sc as plsc

_VOCAB = 1000000
_DIM = 32
_BATCH = 16384
_NEG = 5
_ALPHA = 0.025

_NC = 2
_NS = 16
_NW = _NC * _NS
_CPW = _BATCH // _NW
_CPS = _BATCH // _NS
_DUMP = _BATCH
_NDUMP = 1024

_f32 = jnp.float32
_i32 = jnp.int32


def _gather_body(ev, ec, u2, p2, n3, U, P, N, idxb, rowb):
  wid = lax.axis_index("s") * _NC + lax.axis_index("c")
  r0 = wid * (_CPW // 128)
  b0 = wid * _CPW

  def fetch(tbl, idx_src, out_dst):
    pltpu.sync_copy(idx_src, idxb)
    for j in range(_CPW // 128):
      pltpu.sync_copy(tbl.at[idxb.at[j]], rowb.at[pl.ds(j * 128, 128)])
    pltpu.sync_copy(rowb, out_dst)

  fetch(ev, u2.at[pl.ds(r0, _CPW // 128)], U.at[pl.ds(b0, _CPW)])
  fetch(ec, p2.at[pl.ds(r0, _CPW // 128)], P.at[pl.ds(b0, _CPW)])
  for k in range(_NEG):
    fetch(ec, n3.at[k, pl.ds(r0, _CPW // 128)], N.at[k, pl.ds(b0, _CPW)])


_gather_call = pl.kernel(
    _gather_body,
    out_type=(
        jax.ShapeDtypeStruct((_BATCH, _DIM), _f32),
        jax.ShapeDtypeStruct((_BATCH, _DIM), _f32),
        jax.ShapeDtypeStruct((_NEG, _BATCH, _DIM), _f32),
    ),
    mesh=plsc.VectorSubcoreMesh(core_axis_name="c", subcore_axis_name="s",
                                num_cores=_NC, num_subcores=_NS),
    scratch_types=[
        pltpu.VMEM((_CPW // 128, 128), _i32),
        pltpu.VMEM((_CPW, _DIM), _f32),
    ],
    compiler_params=pltpu.CompilerParams(use_tc_tiling_on_sc=False),
)


def _err_body(u_ref, p_ref, n_ref, e_ref):
  uu = u_ref[...]
  pp = p_ref[...]
  sp = jnp.sum(uu * pp, axis=1)
  e = (_ALPHA * (1.0 - jax.nn.sigmoid(sp)))[:, None] * pp
  for k in range(_NEG):
    nk = n_ref[k]
    sk = jnp.sum(uu * nk, axis=1)
    e = e - (_ALPHA * jax.nn.sigmoid(sk))[:, None] * nk
  e_ref[...] = e


_BB = 2048

_err_call = pl.pallas_call(
    _err_body,
    grid=(_BATCH // _BB,),
    in_specs=[
        pl.BlockSpec((_BB, _DIM), lambda i: (i, 0)),
        pl.BlockSpec((_BB, _DIM), lambda i: (i, 0)),
        pl.BlockSpec((_NEG, _BB, _DIM), lambda i: (0, i, 0)),
    ],
    out_specs=pl.BlockSpec((_BB, _DIM), lambda i: (i, 0)),
    out_shape=jax.ShapeDtypeStruct((_BATCH, _DIM), _f32),
)


_TW = 16


def _scatter_body(out_hbm, err_hbm, u2, bv2, tagtab,
                  idxb, bvb, wb, wsc, tbuf, errb, rowb, delta):
  wid = lax.axis_index("s")
  r0 = wid * (_CPS // 128)
  b0 = wid * _CPS
  nj = _CPS // 128
  io = lax.broadcasted_iota(_i32, (16,), 0)
  zero = jnp.zeros((16,), _i32)

  pltpu.sync_copy(u2.at[pl.ds(r0, nj)], idxb)
  pltpu.sync_copy(bv2.at[pl.ds(r0, nj)], bvb)
  pltpu.sync_copy(err_hbm.at[pl.ds(b0, _CPS)], errb)
  pltpu.sync_copy(errb, delta.at[pl.ds(b0, _CPS)])

  for j in range(nj):
    for cc in range(8):
      rid = io + (j * 128 + cc * 16)
      plsc.store_scatter(tbuf, [rid, zero], bvb[j, pl.ds(cc * 16, 16)])
  for j in range(nj):
    pltpu.sync_copy(tbuf.at[pl.ds(j * 128, 128)], tagtab.at[idxb.at[j]])

  plsc.subcore_barrier()

  for j in range(nj):
    pltpu.sync_copy(tagtab.at[idxb.at[j]], tbuf.at[pl.ds(j * 128, 128)])
  for j in range(nj):
    for cc in range(8):
      sl = pl.ds(cc * 16, 16)
      rid = io + (j * 128 + cc * 16)
      wv = plsc.load_gather(tbuf, [rid, zero])
      bvv = bvb[j, sl]
      wb[j, sl] = wv
      wsc[j, sl] = jnp.where(wv == bvv, _DUMP + (bvv & (_NDUMP - 1)), wv)
  for j in range(nj):
    pltpu.sync_copy(errb.at[pl.ds(j * 128, 128)], delta.at[wsc.at[j]],
                    add=True)
  for j in range(nj):
    pltpu.sync_copy(out_hbm.at[idxb.at[j]], rowb.at[pl.ds(j * 128, 128)])
  for j in range(nj):
    pltpu.sync_copy(rowb.at[pl.ds(j * 128, 128)], delta.at[bvb.at[j]],
                    add=True)

  plsc.subcore_barrier()

  for j in range(nj):
    pltpu.sync_copy(delta.at[wb.at[j]], errb.at[pl.ds(j * 128, 128)])
  for j in range(nj):
    pltpu.sync_copy(errb.at[pl.ds(j * 128, 128)], out_hbm.at[idxb.at[j]])


_scatter_call = pl.kernel(
    _scatter_body,
    out_type=jax.ShapeDtypeStruct((_VOCAB, _TW), _i32),
    mesh=plsc.VectorSubcoreMesh(core_axis_name="c", subcore_axis_name="s",
                                num_cores=1, num_subcores=_NS),
    scratch_types=[
        pltpu.VMEM((_CPS // 128, 128), _i32),
        pltpu.VMEM((_CPS // 128, 128), _i32),
        pltpu.VMEM((_CPS // 128, 128), _i32),
        pltpu.VMEM((_CPS // 128, 128), _i32),
        pltpu.VMEM((_CPS, _TW), _i32),
        pltpu.VMEM((_CPS, _DIM), _f32),
        pltpu.VMEM((_CPS, _DIM), _f32),
        pltpu.VMEM_SHARED((_BATCH + _NDUMP, _DIM), _f32),
    ],
    compiler_params=pltpu.CompilerParams(use_tc_tiling_on_sc=False,
                                         needs_layout_passes=False),
)


def kernel(emb_vertex, emb_context, u, pos_v, neg_v):
  u2 = u.reshape(128, 128)
  p2 = pos_v.reshape(128, 128)
  n3 = neg_v.T.reshape(_NEG, 128, 128)
  bv2 = jnp.arange(_BATCH, dtype=_i32).reshape(128, 128)

  out_ref = jax.new_ref(emb_vertex)
  U, P, N = _gather_call(out_ref, emb_context, u2, p2, n3)
  err = _err_call(U, P, N)

  _scatter_call(out_ref, err, u2, bv2)
  return out_ref[...]
